# baseline (speedup 1.0000x reference)
import jax
import jax.numpy as jnp
from jax import lax
from jax.experimental import pallas as pl
from jax.experimental.pallas import tpu as pltpu
from jax.experimental.pallas import tpu_sc as plsc

NC, NS, L = 2, 16, 16
NW = NC * NS
B, S, D = 1024, 200, 128
LSEQ = S + 8
ND = D // L
NB = B // NW
HALF = S // 2
EPS = 1e-12


def _rsqrt(v):
    i = lax.bitcast_convert_type(v, jnp.int32)
    r = lax.bitcast_convert_type(jnp.int32(0x5F3759DF) - (i >> 1), jnp.float32)
    for _ in range(1):
        r = r * (1.5 - 0.5 * v * r * r)
    return r


def _tree8(v):
    return ((v[0] + v[1]) + (v[2] + v[3])) + ((v[4] + v[5]) + (v[6] + v[7]))


def _hsum(v):
    idx = lax.iota(jnp.int32, L)
    dnums = lax.GatherDimensionNumbers(offset_dims=(), collapsed_slice_dims=(0,),
                                       start_index_map=(0,))
    for sh in (8, 4, 2, 1):
        perm = jnp.bitwise_xor(idx, sh).reshape(L, 1)
        v = v + lax.gather(v, perm, dnums, (1,),
                           mode=lax.GatherScatterMode.PROMISE_IN_BOUNDS)
    return v


_DNUMS = lax.GatherDimensionNumbers(offset_dims=(), collapsed_slice_dims=(0,),
                                    start_index_map=(0,))


def _perm(v, pv):
    return lax.gather(v, pv, _DNUMS, (1,),
                      mode=lax.GatherScatterMode.PROMISE_IN_BOUNDS)


def _ln_row(xs, gam_v, bet_v):
    tot = _hsum(_tree8(xs))
    ssq = _hsum(_tree8([x * x for x in xs]))
    mean = tot * (1.0 / D)
    var = jnp.maximum(ssq * (1.0 / D) - mean * mean, 0.0)
    a = _rsqrt(var + EPS)
    ys = []
    for d in range(ND):
        g = gam_v[pl.ds(d * L, L)]
        t = bet_v[pl.ds(d * L, L)]
        ys.append((xs[d] - mean) * a * g + t)
    return ys


def _body(wt, ids, pre, suf, pos, gam, bet, out,
          b0, b1, b2, pos_v, ids_v, psrc_v, gam_v, bet_v,
          g0, g1, g2, o0, o1, o2):
    bufs = (b0, b1, b2)
    gsems = (g0, g1, g2)
    osems = (o0, o1, o2)
    wid = lax.axis_index("s") * NC + lax.axis_index("c")
    base_b = wid * NB

    pltpu.sync_copy(pos.at[pl.ds(0, LSEQ)], pos_v)
    pltpu.sync_copy(ids.at[pl.ds(wid * (NB * 2), NB * 2)], ids_v)
    pltpu.sync_copy(gam, gam_v)
    pltpu.sync_copy(bet, bet_v)
    pltpu.sync_copy(pre, psrc_v.at[pl.ds(0, 4)])
    pltpu.sync_copy(suf, psrc_v.at[pl.ds(4, 4)])

    @pl.loop(0, 8)
    def _ps(j):
        drow = jnp.where(j < 4, j, j + S)
        xs = [psrc_v[j, pl.ds(d * L, L)] + pos_v[drow, pl.ds(d * L, L)]
              for d in range(ND)]
        ys = _ln_row(xs, gam_v, bet_v)
        for d in range(ND):
            b0[drow, pl.ds(d * L, L)] = ys[d]
            b1[drow, pl.ds(d * L, L)] = ys[d]
            b2[drow, pl.ds(d * L, L)] = ys[d]

    def issue_gather(i, p):
        @pl.when(i < NB)
        def _():
            pltpu.async_copy(wt.at[ids_v.at[2 * i]],
                             bufs[p].at[pl.ds(4, HALF)], gsems[p])
            pltpu.async_copy(wt.at[ids_v.at[2 * i + 1]],
                             bufs[p].at[pl.ds(4 + HALF, HALF)], gsems[p])

    def wait_gather(p):
        pltpu.make_async_copy(wt.at[ids_v.at[0]],
                              bufs[p].at[pl.ds(4, HALF)], gsems[p]).wait()
        pltpu.make_async_copy(wt.at[ids_v.at[0]],
                              bufs[p].at[pl.ds(4 + HALF, HALF)], gsems[p]).wait()

    def issue_out(i, p):
        pltpu.async_copy(bufs[p], out.at[base_b + i], osems[p])

    def wait_out(p):
        pltpu.make_async_copy(bufs[p], out.at[base_b], osems[p]).wait()

    lane = lax.iota(jnp.int32, L)
    p8 = jnp.bitwise_xor(lane, 8).reshape(L, 1)
    p4 = jnp.bitwise_xor(lane, 4).reshape(L, 1)
    p2 = jnp.bitwise_xor(lane, 2).reshape(L, 1)
    p1 = jnp.bitwise_xor(lane, 1).reshape(L, 1)
    zq = [jnp.full((L, 1), 4 * k, jnp.int32) for k in range(4)]
    m4a = lane < 4
    m8 = lane < 8
    m4b = lane < 12

    gs = [gam_v[pl.ds(d * L, L)] for d in range(ND)]
    ts = [bet_v[pl.ds(d * L, L)] for d in range(ND)]

    def compute(p):
        buf = bufs[p]

        def quad(r):
            ss = []
            qs = []
            for k in range(4):
                xk = [buf[r + k, pl.ds(d * L, L)] + pos_v[r + k, pl.ds(d * L, L)]
                      for d in range(ND)]
                for d in range(ND):
                    buf[r + k, pl.ds(d * L, L)] = xk[d]
                s = _tree8(xk)
                q = _tree8([x * x for x in xk])
                s = s + _perm(s, p8)
                q = q + _perm(q, p8)
                s = s + _perm(s, p4)
                q = q + _perm(q, p4)
                ss.append(s)
                qs.append(q)
            sp = jnp.where(m8, jnp.where(m4a, ss[0], ss[1]),
                           jnp.where(m4b, ss[2], ss[3]))
            qp = jnp.where(m8, jnp.where(m4a, qs[0], qs[1]),
                           jnp.where(m4b, qs[2], qs[3]))
            for pv in (p2, p1):
                sp = sp + _perm(sp, pv)
                qp = qp + _perm(qp, pv)
            meanp = sp * (1.0 / D)
            varp = jnp.maximum(qp * (1.0 / D) - meanp * meanp, 0.0)
            ap = _rsqrt(varp + EPS)
            for k in range(4):
                mk = _perm(meanp, zq[k])
                ak = _perm(ap, zq[k])
                for d in range(ND):
                    buf[r + k, pl.ds(d * L, L)] = \
                        (buf[r + k, pl.ds(d * L, L)] - mk) * ak * gs[d] + ts[d]

        @pl.loop(4, 4 + S, step=20)
        def _row(r):
            for j in range(5):
                quad(r + 4 * j)

    def step(i, p):
        wait_gather(p)
        compute(p)
        issue_out(i, p)
        q = (p + 2) % 3

        @pl.when(i >= 1)
        def _():
            wait_out(q)

        issue_gather(i + 2, q)

    issue_gather(0, 0)
    issue_gather(1, 1)

    @pl.loop(0, NB - 2, step=3)
    def _main(g):
        step(g, 0)
        step(g + 1, 1)
        step(g + 2, 2)

    step(NB - 2, 0)
    step(NB - 1, 1)
    wait_out(1)


@jax.jit
def kernel(input_ids, word_table, prefix_table, suffix_table, pos_table,
           ln_gamma, ln_beta):
    ids2 = input_ids.astype(jnp.int32).reshape(B * S // HALF, HALF)
    f = pl.kernel(
        _body,
        out_type=jax.ShapeDtypeStruct((B, LSEQ, D), jnp.float32),
        mesh=plsc.VectorSubcoreMesh(core_axis_name="c", subcore_axis_name="s",
                                    num_cores=NC, num_subcores=NS),
        scratch_types=[
            pltpu.VMEM((LSEQ, D), jnp.float32),
            pltpu.VMEM((LSEQ, D), jnp.float32),
            pltpu.VMEM((LSEQ, D), jnp.float32),
            pltpu.VMEM((LSEQ, D), jnp.float32),
            pltpu.VMEM((NB * 2, HALF), jnp.int32),
            pltpu.VMEM((8, D), jnp.float32),
            pltpu.VMEM((D,), jnp.float32),
            pltpu.VMEM((D,), jnp.float32),
            pltpu.SemaphoreType.DMA,
            pltpu.SemaphoreType.DMA,
            pltpu.SemaphoreType.DMA,
            pltpu.SemaphoreType.DMA,
            pltpu.SemaphoreType.DMA,
            pltpu.SemaphoreType.DMA,
        ],
    )
    return f(word_table, ids2, prefix_table, suffix_table, pos_table,
             ln_gamma, ln_beta)


# speedup vs baseline: 1.2242x; 1.2242x over previous
import jax
import jax.numpy as jnp
from jax import lax
from jax.experimental import pallas as pl
from jax.experimental.pallas import tpu as pltpu
from jax.experimental.pallas import tpu_sc as plsc

NC, NS, L = 2, 16, 16
NW = NC * NS
B, S, D = 1024, 200, 128
LSEQ = S + 8
ND = D // L
NB = B // NW
HALF = S // 2
EPS = 1e-12


def _rsqrt(v):
    i = lax.bitcast_convert_type(v, jnp.int32)
    r = lax.bitcast_convert_type(jnp.int32(0x5F3759DF) - (i >> 1), jnp.float32)
    for _ in range(1):
        r = r * (1.5 - 0.5 * v * r * r)
    return r


def _tree8(v):
    return ((v[0] + v[1]) + (v[2] + v[3])) + ((v[4] + v[5]) + (v[6] + v[7]))


def _hsum(v):
    idx = lax.iota(jnp.int32, L)
    dnums = lax.GatherDimensionNumbers(offset_dims=(), collapsed_slice_dims=(0,),
                                       start_index_map=(0,))
    for sh in (8, 4, 2, 1):
        perm = jnp.bitwise_xor(idx, sh).reshape(L, 1)
        v = v + lax.gather(v, perm, dnums, (1,),
                           mode=lax.GatherScatterMode.PROMISE_IN_BOUNDS)
    return v


_DNUMS = lax.GatherDimensionNumbers(offset_dims=(), collapsed_slice_dims=(0,),
                                    start_index_map=(0,))


def _perm(v, pv):
    return lax.gather(v, pv, _DNUMS, (1,),
                      mode=lax.GatherScatterMode.PROMISE_IN_BOUNDS)


def _ln_row(xs, gam_v, bet_v):
    tot = _hsum(_tree8(xs))
    ssq = _hsum(_tree8([x * x for x in xs]))
    mean = tot * (1.0 / D)
    var = jnp.maximum(ssq * (1.0 / D) - mean * mean, 0.0)
    a = _rsqrt(var + EPS)
    ys = []
    for d in range(ND):
        g = gam_v[pl.ds(d * L, L)]
        t = bet_v[pl.ds(d * L, L)]
        ys.append((xs[d] - mean) * a * g + t)
    return ys


def _body(wt, ids, pre, suf, pos, gam, bet, out,
          b0, b1, b2, pos_v, ids_v, psrc_v, gam_v, bet_v,
          g0, g1, g2, o0, o1, o2):
    bufs = (b0, b1, b2)
    gsems = (g0, g1, g2)
    osems = (o0, o1, o2)
    wid = lax.axis_index("s") * NC + lax.axis_index("c")
    base_b = wid * NB

    pltpu.sync_copy(pos.at[pl.ds(0, LSEQ)], pos_v)
    pltpu.sync_copy(ids.at[pl.ds(wid * (NB * 2), NB * 2)], ids_v)
    pltpu.sync_copy(gam, gam_v)
    pltpu.sync_copy(bet, bet_v)
    pltpu.sync_copy(pre, psrc_v.at[pl.ds(0, 4)])
    pltpu.sync_copy(suf, psrc_v.at[pl.ds(4, 4)])

    @pl.loop(0, 8)
    def _ps(j):
        drow = jnp.where(j < 4, j, j + S)
        xs = [psrc_v[j, pl.ds(d * L, L)] + pos_v[drow, pl.ds(d * L, L)]
              for d in range(ND)]
        ys = _ln_row(xs, gam_v, bet_v)
        for d in range(ND):
            b0[drow, pl.ds(d * L, L)] = ys[d]
            b1[drow, pl.ds(d * L, L)] = ys[d]
            b2[drow, pl.ds(d * L, L)] = ys[d]

    def issue_gather(i, p):
        @pl.when(i < NB)
        def _():
            pltpu.async_copy(wt.at[ids_v.at[2 * i]],
                             bufs[p].at[pl.ds(4, HALF)], gsems[p])
            pltpu.async_copy(wt.at[ids_v.at[2 * i + 1]],
                             bufs[p].at[pl.ds(4 + HALF, HALF)], gsems[p])

    def wait_gather(p):
        pltpu.make_async_copy(wt.at[ids_v.at[0]],
                              bufs[p].at[pl.ds(4, HALF)], gsems[p]).wait()
        pltpu.make_async_copy(wt.at[ids_v.at[0]],
                              bufs[p].at[pl.ds(4 + HALF, HALF)], gsems[p]).wait()

    def issue_out(i, p):
        pltpu.async_copy(bufs[p], out.at[base_b + i], osems[p])

    def wait_out(p):
        pltpu.make_async_copy(bufs[p], out.at[base_b], osems[p]).wait()

    lane = lax.iota(jnp.int32, L)
    p8 = jnp.bitwise_xor(lane, 8).reshape(L, 1)
    p4 = jnp.bitwise_xor(lane, 4).reshape(L, 1)
    p2 = jnp.bitwise_xor(lane, 2).reshape(L, 1)
    p1 = jnp.bitwise_xor(lane, 1).reshape(L, 1)
    zq = [jnp.full((L, 1), 4 * k, jnp.int32) for k in range(4)]
    m4a = lane < 4
    m8 = lane < 8
    m4b = lane < 12

    gs = [gam_v[pl.ds(d * L, L)] for d in range(ND)]
    ts = [bet_v[pl.ds(d * L, L)] for d in range(ND)]

    def compute(p):
        buf = bufs[p]

        def quad(r):
            ss = []
            qs = []
            for k in range(4):
                xk = [buf[r + k, pl.ds(d * L, L)] + pos_v[r + k, pl.ds(d * L, L)]
                      for d in range(ND)]
                for d in range(ND):
                    buf[r + k, pl.ds(d * L, L)] = xk[d]
                s = _tree8(xk)
                q = _tree8([x * x for x in xk])
                s = s + _perm(s, p8)
                q = q + _perm(q, p8)
                s = s + _perm(s, p4)
                q = q + _perm(q, p4)
                ss.append(s)
                qs.append(q)
            sp = jnp.where(m8, jnp.where(m4a, ss[0], ss[1]),
                           jnp.where(m4b, ss[2], ss[3]))
            qp = jnp.where(m8, jnp.where(m4a, qs[0], qs[1]),
                           jnp.where(m4b, qs[2], qs[3]))
            for pv in (p2, p1):
                sp = sp + _perm(sp, pv)
                qp = qp + _perm(qp, pv)
            meanp = sp * (1.0 / D)
            varp = jnp.maximum(qp * (1.0 / D) - meanp * meanp, 0.0)
            ap = _rsqrt(varp + EPS)
            for k in range(4):
                mk = _perm(meanp, zq[k])
                ak = _perm(ap, zq[k])
                for d in range(ND):
                    buf[r + k, pl.ds(d * L, L)] = \
                        (buf[r + k, pl.ds(d * L, L)] - mk) * ak * gs[d] + ts[d]

        @pl.loop(4, 4 + S, step=8)
        def _row(r):
            quad(r)
            quad(r + 4)

    def step(i, p):
        wait_gather(p)
        compute(p)
        issue_out(i, p)
        q = (p + 2) % 3

        @pl.when(i >= 1)
        def _():
            wait_out(q)

        issue_gather(i + 2, q)

    issue_gather(0, 0)
    issue_gather(1, 1)

    @pl.loop(0, NB - 2, step=3)
    def _main(g):
        step(g, 0)
        step(g + 1, 1)
        step(g + 2, 2)

    step(NB - 2, 0)
    step(NB - 1, 1)
    wait_out(1)


@jax.jit
def kernel(input_ids, word_table, prefix_table, suffix_table, pos_table,
           ln_gamma, ln_beta):
    ids2 = input_ids.astype(jnp.int32).reshape(B * S // HALF, HALF)
    f = pl.kernel(
        _body,
        out_type=jax.ShapeDtypeStruct((B, LSEQ, D), jnp.float32),
        mesh=plsc.VectorSubcoreMesh(core_axis_name="c", subcore_axis_name="s",
                                    num_cores=NC, num_subcores=NS),
        scratch_types=[
            pltpu.VMEM((LSEQ, D), jnp.float32),
            pltpu.VMEM((LSEQ, D), jnp.float32),
            pltpu.VMEM((LSEQ, D), jnp.float32),
            pltpu.VMEM((LSEQ, D), jnp.float32),
            pltpu.VMEM((NB * 2, HALF), jnp.int32),
            pltpu.VMEM((8, D), jnp.float32),
            pltpu.VMEM((D,), jnp.float32),
            pltpu.VMEM((D,), jnp.float32),
            pltpu.SemaphoreType.DMA,
            pltpu.SemaphoreType.DMA,
            pltpu.SemaphoreType.DMA,
            pltpu.SemaphoreType.DMA,
            pltpu.SemaphoreType.DMA,
            pltpu.SemaphoreType.DMA,
        ],
    )
    return f(word_table, ids2, prefix_table, suffix_table, pos_table,
             ln_gamma, ln_beta)
